# Initial kernel scaffold; baseline (speedup 1.0000x reference)
import jax
import jax.numpy as jnp
from jax.experimental import pallas as pl


def kernel(embeddings, edge_index, edge_attr, w1, b1, w2, b2):
    raise NotImplementedError("write your pallas kernel here")


# speedup vs baseline: 1.3623x; 1.3623x over previous
import jax
import jax.numpy as jnp
from jax import lax
from jax.experimental import pallas as pl
from jax.experimental.pallas import tpu as pltpu

_LANE = 128
_TILE_E = 2048
_VMEM_LIMIT = 48 << 20


def _round_up(x, m):
    return ((x + m - 1) // m) * m


def _edge_kernel(idx_ref, attr_ref, a_src_ref, a_dst_ref,
                 w1a_ref, w2_ref, b2_ref, o_ref):
    n_pad = a_src_ref.shape[1]
    t = idx_ref.shape[1]
    idx = idx_ref[...]
    node_iota = lax.broadcasted_iota(jnp.int32, (n_pad, t), 0)
    oh_src = (node_iota == idx[0:1, :]).astype(jnp.bfloat16)
    oh_dst = (node_iota == idx[1:2, :]).astype(jnp.bfloat16)
    h = jnp.dot(a_src_ref[...], oh_src, preferred_element_type=jnp.float32)
    h = h + jnp.dot(a_dst_ref[...], oh_dst, preferred_element_type=jnp.float32)
    h = h + jnp.dot(w1a_ref[...], attr_ref[...],
                    preferred_element_type=jnp.float32)
    h = jnp.tanh(h)
    y = jnp.dot(w2_ref[...], h, preferred_element_type=jnp.float32)
    o_ref[...] = jax.nn.sigmoid(y + b2_ref[...])


def kernel(embeddings, edge_index, edge_attr, w1, b1, w2, b2):
    num_nodes, emb_dim = embeddings.shape
    num_edges, attr_dim = edge_attr.shape
    d, hidden = w1.shape

    n_pad = _round_up(max(num_nodes, 1), 8)

    w1f = w1.astype(jnp.float32)
    embf = embeddings.astype(jnp.float32)
    a_src = (embf @ w1f[:emb_dim]).T + b1.reshape(hidden, 1).astype(jnp.float32)
    a_dst = (embf @ w1f[emb_dim:2 * emb_dim]).T
    if n_pad != num_nodes:
        pad = ((0, 0), (0, n_pad - num_nodes))
        a_src = jnp.pad(a_src, pad)
        a_dst = jnp.pad(a_dst, pad)
    a_src = a_src.astype(jnp.bfloat16)
    a_dst = a_dst.astype(jnp.bfloat16)
    w1_attr_t = w1f[2 * emb_dim:].T.astype(jnp.bfloat16)
    w2_t = w2.T.astype(jnp.float32)
    b2_c = b2.reshape(1, 1).astype(jnp.float32)

    tile = min(_TILE_E, _round_up(max(num_edges, 1), _LANE))
    n_tiles = pl.cdiv(num_edges, tile)
    e_pad = n_tiles * tile

    idx = edge_index.astype(jnp.int32)
    attr_t = edge_attr.astype(jnp.bfloat16).T
    if e_pad != num_edges:
        idx = jnp.pad(idx, ((0, 0), (0, e_pad - num_edges)))
        attr_t = jnp.pad(attr_t, ((0, 0), (0, e_pad - num_edges)))

    out = pl.pallas_call(
        _edge_kernel,
        out_shape=jax.ShapeDtypeStruct((1, e_pad), jnp.float32),
        grid_spec=pltpu.PrefetchScalarGridSpec(
            num_scalar_prefetch=0,
            grid=(n_tiles,),
            in_specs=[
                pl.BlockSpec((2, tile), lambda i: (0, i)),
                pl.BlockSpec((attr_dim, tile), lambda i: (0, i)),
                pl.BlockSpec((hidden, n_pad), lambda i: (0, 0)),
                pl.BlockSpec((hidden, n_pad), lambda i: (0, 0)),
                pl.BlockSpec((hidden, attr_dim), lambda i: (0, 0)),
                pl.BlockSpec((1, hidden), lambda i: (0, 0)),
                pl.BlockSpec((1, 1), lambda i: (0, 0)),
            ],
            out_specs=pl.BlockSpec((1, tile), lambda i: (0, i)),
        ),
        compiler_params=pltpu.CompilerParams(
            dimension_semantics=("parallel",),
            vmem_limit_bytes=_VMEM_LIMIT,
        ),
    )(idx, attr_t, a_src, a_dst, w1_attr_t, w2_t, b2_c)
    return out[:, :num_edges].T


# speedup vs baseline: 1.9313x; 1.4177x over previous
import jax
import jax.numpy as jnp
from jax import lax
from jax.experimental import pallas as pl
from jax.experimental.pallas import tpu as pltpu

_LANE = 128
_TILE_E = 2048
_VMEM_LIMIT = 48 << 20


def _round_up(x, m):
    return ((x + m - 1) // m) * m


def _edge_kernel(idx_ref, attr_ref, a_src_ref, a_dst_ref,
                 w1a_ref, w2_ref, b2_ref, o_ref):
    n_pad = a_src_ref.shape[1]
    t = idx_ref.shape[1]
    idx = idx_ref[...]
    node_iota = lax.broadcasted_iota(jnp.int32, (n_pad, t), 0)
    oh_src = (node_iota == idx[0:1, :]).astype(jnp.bfloat16)
    oh_dst = (node_iota == idx[1:2, :]).astype(jnp.bfloat16)
    h = jnp.dot(a_src_ref[...], oh_src, preferred_element_type=jnp.float32)
    h = h + jnp.dot(a_dst_ref[...], oh_dst, preferred_element_type=jnp.float32)
    attr_b = attr_ref[...].astype(jnp.bfloat16)
    h = h + lax.dot_general(w1a_ref[...], attr_b,
                            (((1,), (1,)), ((), ())),
                            preferred_element_type=jnp.float32)
    h = jnp.tanh(h)
    y = jnp.dot(w2_ref[...], h, preferred_element_type=jnp.float32)
    o_ref[...] = jax.nn.sigmoid(y + b2_ref[...])


def kernel(embeddings, edge_index, edge_attr, w1, b1, w2, b2):
    num_nodes, emb_dim = embeddings.shape
    num_edges, attr_dim = edge_attr.shape
    d, hidden = w1.shape

    n_pad = _round_up(max(num_nodes, 1), 8)

    w1f = w1.astype(jnp.float32)
    embf = embeddings.astype(jnp.float32)
    a_src = (embf @ w1f[:emb_dim]).T + b1.reshape(hidden, 1).astype(jnp.float32)
    a_dst = (embf @ w1f[emb_dim:2 * emb_dim]).T
    if n_pad != num_nodes:
        pad = ((0, 0), (0, n_pad - num_nodes))
        a_src = jnp.pad(a_src, pad)
        a_dst = jnp.pad(a_dst, pad)
    a_src = a_src.astype(jnp.bfloat16)
    a_dst = a_dst.astype(jnp.bfloat16)
    w1_attr_t = w1f[2 * emb_dim:].T.astype(jnp.bfloat16)
    w2_t = w2.T.astype(jnp.float32)
    b2_c = b2.reshape(1, 1).astype(jnp.float32)

    tile = min(_TILE_E, _round_up(max(num_edges, 1), _LANE))
    n_tiles = pl.cdiv(num_edges, tile)
    e_pad = n_tiles * tile

    idx = edge_index.astype(jnp.int32)
    attr = edge_attr.astype(jnp.float32)
    if e_pad != num_edges:
        idx = jnp.pad(idx, ((0, 0), (0, e_pad - num_edges)))
        attr = jnp.pad(attr, ((0, e_pad - num_edges), (0, 0)))

    out = pl.pallas_call(
        _edge_kernel,
        out_shape=jax.ShapeDtypeStruct((1, e_pad), jnp.float32),
        grid_spec=pltpu.PrefetchScalarGridSpec(
            num_scalar_prefetch=0,
            grid=(n_tiles,),
            in_specs=[
                pl.BlockSpec((2, tile), lambda i: (0, i)),
                pl.BlockSpec((tile, attr_dim), lambda i: (i, 0)),
                pl.BlockSpec((hidden, n_pad), lambda i: (0, 0)),
                pl.BlockSpec((hidden, n_pad), lambda i: (0, 0)),
                pl.BlockSpec((hidden, attr_dim), lambda i: (0, 0)),
                pl.BlockSpec((1, hidden), lambda i: (0, 0)),
                pl.BlockSpec((1, 1), lambda i: (0, 0)),
            ],
            out_specs=pl.BlockSpec((1, tile), lambda i: (0, i)),
        ),
        compiler_params=pltpu.CompilerParams(
            dimension_semantics=("parallel",),
            vmem_limit_bytes=_VMEM_LIMIT,
        ),
    )(idx, attr, a_src, a_dst, w1_attr_t, w2_t, b2_c)
    if e_pad != num_edges:
        out = out[:, :num_edges]
    return jnp.reshape(out, (num_edges, 1))


# speedup vs baseline: 2.1476x; 1.1120x over previous
import jax
import jax.numpy as jnp
from jax import lax
from jax.experimental import pallas as pl
from jax.experimental.pallas import tpu as pltpu

_LANE = 128
_TILE_E = 4096
_VMEM_LIMIT = 58 << 20


def _round_up(x, m):
    return ((x + m - 1) // m) * m


def _edge_kernel(idx_ref, attr_ref, a_src_ref, a_dst_ref,
                 w1a_ref, w2_ref, b2_ref, o_ref):
    n_pad = a_src_ref.shape[1]
    t = idx_ref.shape[1]
    idx = idx_ref[...]
    node_iota = lax.broadcasted_iota(jnp.int32, (n_pad, t), 0)
    oh_src = (node_iota == idx[0:1, :]).astype(jnp.bfloat16)
    oh_dst = (node_iota == idx[1:2, :]).astype(jnp.bfloat16)
    h = jnp.dot(a_src_ref[...], oh_src, preferred_element_type=jnp.float32)
    h = h + jnp.dot(a_dst_ref[...], oh_dst, preferred_element_type=jnp.float32)
    attr_b = attr_ref[...].astype(jnp.bfloat16)
    h = h + lax.dot_general(w1a_ref[...], attr_b,
                            (((1,), (1,)), ((), ())),
                            preferred_element_type=jnp.float32)
    h = jnp.tanh(h)
    y = jnp.dot(w2_ref[...], h, preferred_element_type=jnp.float32)
    o_ref[...] = jax.nn.sigmoid(y + b2_ref[...])


def kernel(embeddings, edge_index, edge_attr, w1, b1, w2, b2):
    num_nodes, emb_dim = embeddings.shape
    num_edges, attr_dim = edge_attr.shape
    d, hidden = w1.shape

    n_pad = _round_up(max(num_nodes, 1), 8)

    w1f = w1.astype(jnp.float32)
    embf = embeddings.astype(jnp.float32)
    a_src = (embf @ w1f[:emb_dim]).T + b1.reshape(hidden, 1).astype(jnp.float32)
    a_dst = (embf @ w1f[emb_dim:2 * emb_dim]).T
    if n_pad != num_nodes:
        pad = ((0, 0), (0, n_pad - num_nodes))
        a_src = jnp.pad(a_src, pad)
        a_dst = jnp.pad(a_dst, pad)
    a_src = a_src.astype(jnp.bfloat16)
    a_dst = a_dst.astype(jnp.bfloat16)
    w1_attr_t = w1f[2 * emb_dim:].T.astype(jnp.bfloat16)
    w2_t = w2.T.astype(jnp.float32)
    b2_c = b2.reshape(1, 1).astype(jnp.float32)

    tile = min(_TILE_E, _round_up(max(num_edges, 1), _LANE))
    n_tiles = pl.cdiv(num_edges, tile)
    e_pad = n_tiles * tile

    idx = edge_index.astype(jnp.int32)
    attr = edge_attr.astype(jnp.float32)
    if e_pad != num_edges:
        idx = jnp.pad(idx, ((0, 0), (0, e_pad - num_edges)))
        attr = jnp.pad(attr, ((0, e_pad - num_edges), (0, 0)))

    out = pl.pallas_call(
        _edge_kernel,
        out_shape=jax.ShapeDtypeStruct((1, e_pad), jnp.float32),
        grid_spec=pltpu.PrefetchScalarGridSpec(
            num_scalar_prefetch=0,
            grid=(n_tiles,),
            in_specs=[
                pl.BlockSpec((2, tile), lambda i: (0, i)),
                pl.BlockSpec((tile, attr_dim), lambda i: (i, 0)),
                pl.BlockSpec((hidden, n_pad), lambda i: (0, 0)),
                pl.BlockSpec((hidden, n_pad), lambda i: (0, 0)),
                pl.BlockSpec((hidden, attr_dim), lambda i: (0, 0)),
                pl.BlockSpec((1, hidden), lambda i: (0, 0)),
                pl.BlockSpec((1, 1), lambda i: (0, 0)),
            ],
            out_specs=pl.BlockSpec((1, tile), lambda i: (0, i)),
        ),
        compiler_params=pltpu.CompilerParams(
            dimension_semantics=("parallel",),
            vmem_limit_bytes=_VMEM_LIMIT,
        ),
    )(idx, attr, a_src, a_dst, w1_attr_t, w2_t, b2_c)
    if e_pad != num_edges:
        out = out[:, :num_edges]
    return jnp.reshape(out, (num_edges, 1))


# speedup vs baseline: 2.2080x; 1.0281x over previous
import jax
import jax.numpy as jnp
from jax import lax
from jax.experimental import pallas as pl
from jax.experimental.pallas import tpu as pltpu

_LANE = 128
_TILE_E = 8192
_VMEM_LIMIT = 58 << 20


def _round_up(x, m):
    return ((x + m - 1) // m) * m


def _edge_kernel(idx_ref, attr_ref, a_src_ref, a_dst_ref,
                 w1a_ref, w2_ref, b2_ref, o_ref):
    n_pad = a_src_ref.shape[1]
    t = idx_ref.shape[1]
    idx = idx_ref[...]
    node_iota = lax.broadcasted_iota(jnp.int32, (n_pad, t), 0)
    oh_src = (node_iota == idx[0:1, :]).astype(jnp.bfloat16)
    oh_dst = (node_iota == idx[1:2, :]).astype(jnp.bfloat16)
    h = jnp.dot(a_src_ref[...], oh_src, preferred_element_type=jnp.float32)
    h = h + jnp.dot(a_dst_ref[...], oh_dst, preferred_element_type=jnp.float32)
    attr_b = attr_ref[...].astype(jnp.bfloat16)
    h = h + lax.dot_general(w1a_ref[...], attr_b,
                            (((1,), (1,)), ((), ())),
                            preferred_element_type=jnp.float32)
    h = jnp.tanh(h)
    y = jnp.dot(w2_ref[...], h, preferred_element_type=jnp.float32)
    o_ref[...] = jax.nn.sigmoid(y + b2_ref[...])


def kernel(embeddings, edge_index, edge_attr, w1, b1, w2, b2):
    num_nodes, emb_dim = embeddings.shape
    num_edges, attr_dim = edge_attr.shape
    d, hidden = w1.shape

    n_pad = _round_up(max(num_nodes, 1), 8)

    w1f = w1.astype(jnp.float32)
    embf = embeddings.astype(jnp.float32)
    a_src = (embf @ w1f[:emb_dim]).T + b1.reshape(hidden, 1).astype(jnp.float32)
    a_dst = (embf @ w1f[emb_dim:2 * emb_dim]).T
    if n_pad != num_nodes:
        pad = ((0, 0), (0, n_pad - num_nodes))
        a_src = jnp.pad(a_src, pad)
        a_dst = jnp.pad(a_dst, pad)
    a_src = a_src.astype(jnp.bfloat16)
    a_dst = a_dst.astype(jnp.bfloat16)
    w1_attr_t = w1f[2 * emb_dim:].T.astype(jnp.bfloat16)
    w2_t = w2.T.astype(jnp.float32)
    b2_c = b2.reshape(1, 1).astype(jnp.float32)

    tile = min(_TILE_E, _round_up(max(num_edges, 1), _LANE))
    n_tiles = pl.cdiv(num_edges, tile)
    e_pad = n_tiles * tile

    idx = edge_index.astype(jnp.int32)
    attr = edge_attr.astype(jnp.float32)
    if e_pad != num_edges:
        idx = jnp.pad(idx, ((0, 0), (0, e_pad - num_edges)))
        attr = jnp.pad(attr, ((0, e_pad - num_edges), (0, 0)))

    out = pl.pallas_call(
        _edge_kernel,
        out_shape=jax.ShapeDtypeStruct((1, e_pad), jnp.float32),
        grid_spec=pltpu.PrefetchScalarGridSpec(
            num_scalar_prefetch=0,
            grid=(n_tiles,),
            in_specs=[
                pl.BlockSpec((2, tile), lambda i: (0, i)),
                pl.BlockSpec((tile, attr_dim), lambda i: (i, 0)),
                pl.BlockSpec((hidden, n_pad), lambda i: (0, 0)),
                pl.BlockSpec((hidden, n_pad), lambda i: (0, 0)),
                pl.BlockSpec((hidden, attr_dim), lambda i: (0, 0)),
                pl.BlockSpec((1, hidden), lambda i: (0, 0)),
                pl.BlockSpec((1, 1), lambda i: (0, 0)),
            ],
            out_specs=pl.BlockSpec((1, tile), lambda i: (0, i)),
        ),
        compiler_params=pltpu.CompilerParams(
            dimension_semantics=("parallel",),
            vmem_limit_bytes=_VMEM_LIMIT,
        ),
    )(idx, attr, a_src, a_dst, w1_attr_t, w2_t, b2_c)
    if e_pad != num_edges:
        out = out[:, :num_edges]
    return jnp.reshape(out, (num_edges, 1))


# speedup vs baseline: 2.2107x; 1.0012x over previous
import jax
import jax.numpy as jnp
from jax import lax
from jax.experimental import pallas as pl
from jax.experimental.pallas import tpu as pltpu

_LANE = 128
_TILE_E = 8192
_VMEM_LIMIT = 58 << 20


def _round_up(x, m):
    return ((x + m - 1) // m) * m


def _edge_kernel(idx_ref, attr_ref, acat_ref,
                 w1a_ref, w2_ref, b2_ref, o_ref):
    n_pad = acat_ref.shape[1] // 2
    t = idx_ref.shape[1]
    idx = idx_ref[...]
    pair_iota = (lax.broadcasted_iota(jnp.int32, (n_pad // 2, t), 0)
                 * 131074 + 65536)
    iota16 = pltpu.bitcast(pair_iota, jnp.int16)
    idx_pair = idx * 65537
    idx16 = pltpu.bitcast(idx_pair, jnp.int16)
    one = jnp.ones((), jnp.bfloat16)
    zero = jnp.zeros((), jnp.bfloat16)
    oh_src = jnp.where(iota16 == idx16[0:1, :], one, zero)
    oh_dst = jnp.where(iota16 == idx16[2:3, :], one, zero)
    oh_cat = jnp.concatenate([oh_src, oh_dst], axis=0)
    h = jnp.dot(acat_ref[...], oh_cat, preferred_element_type=jnp.float32)
    attr_b = attr_ref[...].astype(jnp.bfloat16)
    h = h + lax.dot_general(w1a_ref[...], attr_b,
                            (((1,), (1,)), ((), ())),
                            preferred_element_type=jnp.float32)
    h = jnp.tanh(h)
    y = jnp.dot(w2_ref[...], h, preferred_element_type=jnp.float32)
    o_ref[...] = jax.nn.sigmoid(y + b2_ref[...])


def kernel(embeddings, edge_index, edge_attr, w1, b1, w2, b2):
    num_nodes, emb_dim = embeddings.shape
    num_edges, attr_dim = edge_attr.shape
    d, hidden = w1.shape

    n_pad = _round_up(max(num_nodes, 1), 8)

    w1f = w1.astype(jnp.float32)
    embf = embeddings.astype(jnp.float32)
    a_src = (embf @ w1f[:emb_dim]).T + b1.reshape(hidden, 1).astype(jnp.float32)
    a_dst = (embf @ w1f[emb_dim:2 * emb_dim]).T
    if n_pad != num_nodes:
        pad = ((0, 0), (0, n_pad - num_nodes))
        a_src = jnp.pad(a_src, pad)
        a_dst = jnp.pad(a_dst, pad)
    a_cat = jnp.concatenate([a_src, a_dst],
                            axis=1).astype(jnp.bfloat16)
    w1_attr_t = w1f[2 * emb_dim:].T.astype(jnp.bfloat16)
    w2_t = w2.T.astype(jnp.float32)
    b2_c = b2.reshape(1, 1).astype(jnp.float32)

    tile = min(_TILE_E, _round_up(max(num_edges, 1), _LANE))
    n_tiles = pl.cdiv(num_edges, tile)
    e_pad = n_tiles * tile

    idx = edge_index.astype(jnp.int32)
    attr = edge_attr.astype(jnp.float32)
    if e_pad != num_edges:
        idx = jnp.pad(idx, ((0, 0), (0, e_pad - num_edges)))
        attr = jnp.pad(attr, ((0, e_pad - num_edges), (0, 0)))

    out = pl.pallas_call(
        _edge_kernel,
        out_shape=jax.ShapeDtypeStruct((1, e_pad), jnp.float32),
        grid_spec=pltpu.PrefetchScalarGridSpec(
            num_scalar_prefetch=0,
            grid=(n_tiles,),
            in_specs=[
                pl.BlockSpec((2, tile), lambda i: (0, i)),
                pl.BlockSpec((tile, attr_dim), lambda i: (i, 0)),
                pl.BlockSpec((hidden, 2 * n_pad), lambda i: (0, 0)),
                pl.BlockSpec((hidden, attr_dim), lambda i: (0, 0)),
                pl.BlockSpec((1, hidden), lambda i: (0, 0)),
                pl.BlockSpec((1, 1), lambda i: (0, 0)),
            ],
            out_specs=pl.BlockSpec((1, tile), lambda i: (0, i)),
        ),
        compiler_params=pltpu.CompilerParams(
            dimension_semantics=("parallel",),
            vmem_limit_bytes=_VMEM_LIMIT,
        ),
    )(idx, attr, a_cat, w1_attr_t, w2_t, b2_c)
    if e_pad != num_edges:
        out = out[:, :num_edges]
    return jnp.reshape(out, (num_edges, 1))


# speedup vs baseline: 2.2736x; 1.0285x over previous
import jax
import jax.numpy as jnp
from jax import lax
from jax.experimental import pallas as pl
from jax.experimental.pallas import tpu as pltpu

_LANE = 128
_TILE_E = 8192
_VMEM_LIMIT = 58 << 20


def _round_up(x, m):
    return ((x + m - 1) // m) * m


def _edge_kernel(idx_ref, attr_ref, emb_ref, w1_ref, b1_ref,
                 w2_ref, b2_ref, o_ref, acat_ref):
    n_pad, emb_dim = emb_ref.shape
    t = idx_ref.shape[1]

    @pl.when(pl.program_id(0) == 0)
    def _build_tables():
        emb = emb_ref[...]
        a_src = lax.dot_general(w1_ref[0:emb_dim, :], emb,
                                (((0,), (1,)), ((), ())),
                                preferred_element_type=jnp.float32)
        a_dst = lax.dot_general(w1_ref[emb_dim:2 * emb_dim, :], emb,
                                (((0,), (1,)), ((), ())),
                                preferred_element_type=jnp.float32)
        acat_ref[:, 0:n_pad] = (a_src + b1_ref[...]).astype(jnp.bfloat16)
        acat_ref[:, n_pad:2 * n_pad] = a_dst.astype(jnp.bfloat16)

    idx = idx_ref[...]
    pair_iota = (lax.broadcasted_iota(jnp.int32, (n_pad // 2, t), 0)
                 * 131074 + 65536)
    iota16 = pltpu.bitcast(pair_iota, jnp.int16)
    idx_pair = idx * 65537
    idx16 = pltpu.bitcast(idx_pair, jnp.int16)
    one = jnp.ones((), jnp.bfloat16)
    zero = jnp.zeros((), jnp.bfloat16)
    oh_src = jnp.where(iota16 == idx16[0:1, :], one, zero)
    oh_dst = jnp.where(iota16 == idx16[2:3, :], one, zero)
    oh_cat = jnp.concatenate([oh_src, oh_dst], axis=0)
    h = jnp.dot(acat_ref[...], oh_cat, preferred_element_type=jnp.float32)
    attr_b = attr_ref[...].astype(jnp.bfloat16)
    w1a = w1_ref[2 * emb_dim:, :].astype(jnp.bfloat16)
    h = h + lax.dot_general(w1a, attr_b,
                            (((0,), (1,)), ((), ())),
                            preferred_element_type=jnp.float32)
    h = jnp.tanh(h)
    y = jnp.dot(w2_ref[...], h, preferred_element_type=jnp.float32)
    o_ref[...] = jax.nn.sigmoid(y + b2_ref[...])


def kernel(embeddings, edge_index, edge_attr, w1, b1, w2, b2):
    num_nodes, emb_dim = embeddings.shape
    num_edges, attr_dim = edge_attr.shape
    d, hidden = w1.shape

    n_pad = _round_up(max(num_nodes, 1), 16)
    emb = embeddings.astype(jnp.float32)
    if n_pad != num_nodes:
        emb = jnp.pad(emb, ((0, n_pad - num_nodes), (0, 0)))
    w1_c = w1.astype(jnp.float32)
    b1_c = b1.reshape(hidden, 1).astype(jnp.float32)
    w2_t = w2.T.astype(jnp.float32)
    b2_c = b2.reshape(1, 1).astype(jnp.float32)

    tile = min(_TILE_E, _round_up(max(num_edges, 1), _LANE))
    n_tiles = pl.cdiv(num_edges, tile)
    e_pad = n_tiles * tile

    idx = edge_index.astype(jnp.int32)
    attr = edge_attr.astype(jnp.float32)
    if e_pad != num_edges:
        idx = jnp.pad(idx, ((0, 0), (0, e_pad - num_edges)))
        attr = jnp.pad(attr, ((0, e_pad - num_edges), (0, 0)))

    out = pl.pallas_call(
        _edge_kernel,
        out_shape=jax.ShapeDtypeStruct((1, e_pad), jnp.float32),
        grid_spec=pltpu.PrefetchScalarGridSpec(
            num_scalar_prefetch=0,
            grid=(n_tiles,),
            in_specs=[
                pl.BlockSpec((2, tile), lambda i: (0, i)),
                pl.BlockSpec((tile, attr_dim), lambda i: (i, 0)),
                pl.BlockSpec((n_pad, emb_dim), lambda i: (0, 0)),
                pl.BlockSpec((d, hidden), lambda i: (0, 0)),
                pl.BlockSpec((hidden, 1), lambda i: (0, 0)),
                pl.BlockSpec((1, hidden), lambda i: (0, 0)),
                pl.BlockSpec((1, 1), lambda i: (0, 0)),
            ],
            out_specs=pl.BlockSpec((1, tile), lambda i: (0, i)),
            scratch_shapes=[pltpu.VMEM((hidden, 2 * n_pad), jnp.bfloat16)],
        ),
        compiler_params=pltpu.CompilerParams(
            dimension_semantics=("arbitrary",),
            vmem_limit_bytes=_VMEM_LIMIT,
        ),
    )(idx, attr, emb, w1_c, b1_c, w2_t, b2_c)
    if e_pad != num_edges:
        out = out[:, :num_edges]
    return jnp.reshape(out, (num_edges, 1))
